# baseline (speedup 1.0000x reference)
import functools

import jax
import jax.numpy as jnp
from jax import lax
from jax.experimental import pallas as pl
from jax.experimental.pallas import tpu as pltpu
from jax.experimental.pallas import tpu_sc as plsc

NC = 2
NS = 16
NW = NC * NS

_f32 = jnp.float32


def _make_agg_pair_kernel(N, D, E, B2):
    EPW = E // NW
    C = 80
    NCH = EPW // C
    ZR = (N // NS) // 8 * 8
    TAIL = N - NS * ZR
    PPS = B2 // NS
    GC = 64
    NK = PPS // GC
    XPS = B2 // NW
    NX = XPS // GC
    mesh = plsc.VectorSubcoreMesh(core_axis_name="c", subcore_axis_name="s",
                                  num_cores=NC, num_subcores=NS)

    @functools.partial(
        pl.kernel, mesh=mesh,
        out_type=[jax.ShapeDtypeStruct((N, D), _f32),
                  jax.ShapeDtypeStruct((N, D), _f32),
                  jax.ShapeDtypeStruct((N,), _f32),
                  jax.ShapeDtypeStruct((N,), _f32),
                  jax.ShapeDtypeStruct((B2, D), _f32),
                  jax.ShapeDtypeStruct((B2, D), _f32),
                  jax.ShapeDtypeStruct((B2, D), _f32),
                  jax.ShapeDtypeStruct((B2,), _f32),
                  jax.ShapeDtypeStruct((B2,), _f32)],
        scratch_types=[pltpu.VMEM((EPW,), jnp.int32),
                       pltpu.VMEM((NCH, C), jnp.int32),
                       pltpu.VMEM((C,), _f32),
                       pltpu.VMEM((2, C, D), _f32),
                       pltpu.VMEM((PPS,), jnp.int32),
                       pltpu.VMEM((XPS,), jnp.int32),
                       pltpu.VMEM((2, GC), _f32),
                       pltpu.VMEM_SHARED((N, D), _f32),
                       pltpu.VMEM_SHARED((N,), _f32),
                       pltpu.SemaphoreType.DMA,
                       pltpu.SemaphoreType.DMA,
                       pltpu.SemaphoreType.DMA],
    )
    def agg_kernel(x_hbm, src_hbm, dst_hbm, za_hbm, zd_hbm, hp_hbm,
                   a0, a1, d0, d1, ga0, ga1, gx, gd0, gd1,
                   src_v, dst_v, ones_v, rows_v, piv, xiv, sbuf,
                   agg_sh, deg_sh, gsem, ssem, dsem):
        cid = lax.axis_index("c")
        sid = lax.axis_index("s")
        wid = sid * NC + cid
        for k in range(C // 16):
            ones_v[pl.ds(k * 16, 16)] = jnp.ones((16,), _f32)
        pltpu.sync_copy(src_hbm.at[pl.ds(wid * EPW, EPW)], src_v)
        pltpu.sync_copy(dst_hbm.at[wid], dst_v)
        pltpu.sync_copy(hp_hbm.at[pl.ds(sid * PPS, PPS)], piv)
        pltpu.sync_copy(hp_hbm.at[pl.ds(cid * (B2 // NC) + sid * XPS, XPS)],
                        xiv)
        pltpu.sync_copy(za_hbm.at[pl.ds(sid * ZR, ZR), :],
                        agg_sh.at[pl.ds(sid * ZR, ZR), :])

        @pl.when(sid == 0)
        def _():
            pltpu.sync_copy(za_hbm.at[pl.ds(NS * ZR, TAIL), :],
                            agg_sh.at[pl.ds(NS * ZR, TAIL), :])
            pltpu.sync_copy(zd_hbm, deg_sh)

        plsc.subcore_barrier()

        def src_idx(j):
            return src_v.at[pl.ds(pl.multiple_of(j * C, C), C)]

        pltpu.async_copy(x_hbm.at[src_idx(0)], rows_v.at[0], gsem)

        def chunk(j, carry):
            @pl.when(j >= 1)
            def _():
                pltpu.make_async_copy(rows_v.at[(j - 1) % 2],
                                      agg_sh.at[dst_v.at[j - 1]], ssem).wait()
                pltpu.make_async_copy(ones_v,
                                      deg_sh.at[dst_v.at[j - 1]], dsem).wait()

            @pl.when(j + 1 < NCH)
            def _():
                pltpu.async_copy(x_hbm.at[src_idx(j + 1)],
                                 rows_v.at[(j + 1) % 2], gsem)

            pltpu.make_async_copy(x_hbm.at[src_idx(j)],
                                  rows_v.at[j % 2], gsem).wait()
            pltpu.async_copy(rows_v.at[j % 2], agg_sh.at[dst_v.at[j]],
                             ssem, add=True)
            pltpu.async_copy(ones_v, deg_sh.at[dst_v.at[j]], dsem, add=True)
            return carry

        lax.fori_loop(0, NCH, chunk, 0)
        pltpu.make_async_copy(rows_v.at[(NCH - 1) % 2],
                              agg_sh.at[dst_v.at[NCH - 1]], ssem).wait()
        pltpu.make_async_copy(ones_v, deg_sh.at[dst_v.at[NCH - 1]],
                              dsem).wait()
        plsc.subcore_barrier()

        def writeout(a_out, d_out):
            pltpu.sync_copy(agg_sh.at[pl.ds(sid * ZR, ZR), :],
                            a_out.at[pl.ds(sid * ZR, ZR), :])

            @pl.when(sid == 0)
            def _():
                pltpu.sync_copy(agg_sh.at[pl.ds(NS * ZR, TAIL), :],
                                a_out.at[pl.ds(NS * ZR, TAIL), :])
                pltpu.sync_copy(deg_sh, d_out)

        def rbuf(b):
            return rows_v.at[b, pl.ds(0, GC), :]

        def pair_phase(asrc, dsrc, ga, gd):
            def pidx(k):
                return piv.at[pl.ds(pl.multiple_of(k * GC, GC), GC)]

            pltpu.async_copy(asrc.at[pidx(0)], rbuf(0), gsem)
            pltpu.async_copy(dsrc.at[pidx(0)], sbuf.at[0], dsem)
            for k in range(NK):
                if k + 1 < NK:
                    pltpu.async_copy(asrc.at[pidx(k + 1)],
                                     rbuf((k + 1) % 2), gsem)
                    pltpu.async_copy(dsrc.at[pidx(k + 1)],
                                     sbuf.at[(k + 1) % 2], dsem)
                pltpu.make_async_copy(asrc.at[pidx(k)],
                                      rbuf(k % 2), gsem).wait()
                pltpu.make_async_copy(dsrc.at[pidx(k)],
                                      sbuf.at[k % 2], dsem).wait()
                base = sid * PPS + k * GC
                pltpu.sync_copy(rbuf(k % 2), ga.at[pl.ds(base, GC), :])
                pltpu.sync_copy(sbuf.at[k % 2], gd.at[pl.ds(base, GC)])

        def xidx(k):
            return xiv.at[pl.ds(pl.multiple_of(k * GC, GC), GC)]

        def x_phase():
            pltpu.async_copy(x_hbm.at[xidx(0)], rbuf(0), gsem)
            for k in range(NX):
                if k + 1 < NX:
                    pltpu.async_copy(x_hbm.at[xidx(k + 1)],
                                     rbuf((k + 1) % 2), gsem)
                pltpu.make_async_copy(x_hbm.at[xidx(k)],
                                      rbuf(k % 2), gsem).wait()
                base = cid * (B2 // NC) + sid * XPS + k * GC
                pltpu.sync_copy(rbuf(k % 2), gx.at[pl.ds(base, GC), :])

        @pl.when(cid == 0)
        def _():
            writeout(a0, d0)
            plsc.subcore_barrier()
            pair_phase(a0, d0, ga0, gd0)
            x_phase()

        @pl.when(cid == 1)
        def _():
            writeout(a1, d1)
            plsc.subcore_barrier()
            pair_phase(a1, d1, ga1, gd1)
            x_phase()

    return agg_kernel


def _mlp_body(a0h, a0t, a1h, a1t, xh, xt, dh0, dh1, dt0, dt1,
              wg, ws, bg, w1h, w1t, b1r, w2, b2r, w3r, b3r, out_ref):
    def node_repr(a0, a1, xg, da, db):
        agg = a0[...] + a1[...]
        deg = da[...] + db[...]
        s = agg / jnp.maximum(deg, 1.0)
        z = (jnp.dot(s, wg[...], preferred_element_type=_f32)
             + jnp.dot(xg[...], ws[...], preferred_element_type=_f32)
             + bg[...])
        return jnp.maximum(z, 0.0)

    rh = node_repr(a0h, a1h, xh, dh0, dh1)
    rt = node_repr(a0t, a1t, xt, dt0, dt1)
    h = jnp.maximum(jnp.dot(rh, w1h[...], preferred_element_type=_f32)
                    + jnp.dot(rt, w1t[...], preferred_element_type=_f32)
                    + b1r[...], 0.0)
    h = jnp.maximum(jnp.dot(h, w2[...], preferred_element_type=_f32)
                    + b2r[...], 0.0)
    out_ref[...] = jnp.sum(h * w3r[...], axis=1, keepdims=True) + b3r[...]


def _mlp_call(B, D, ga0, ga1, gx, gd0, gd1,
              W_gnn, W_self, bg, W1h, W1t, b1r, W2, b2r, W3r, b3r):
    BLK = 1024
    G = B // BLK
    row_h = pl.BlockSpec((BLK, D), lambda i: (i, 0))
    row_t = pl.BlockSpec((BLK, D), lambda i: (i + G, 0))
    deg_h = pl.BlockSpec((BLK, 1), lambda i: (i, 0))
    deg_t = pl.BlockSpec((BLK, 1), lambda i: (i + G, 0))

    def full(a):
        return pl.BlockSpec(a.shape, lambda i: tuple(0 for _ in a.shape))

    gd0c = gd0.reshape(2 * B, 1)
    gd1c = gd1.reshape(2 * B, 1)
    return pl.pallas_call(
        _mlp_body,
        grid=(G,),
        in_specs=[row_h, row_t, row_h, row_t, row_h, row_t,
                  deg_h, deg_h, deg_t, deg_t,
                  full(W_gnn), full(W_self), full(bg),
                  full(W1h), full(W1t), full(b1r),
                  full(W2), full(b2r), full(W3r), full(b3r)],
        out_specs=pl.BlockSpec((BLK, 1), lambda i: (i, 0)),
        out_shape=jax.ShapeDtypeStruct((B, 1), _f32),
    )(ga0, ga0, ga1, ga1, gx, gx, gd0c, gd1c, gd0c, gd1c,
      W_gnn, W_self, bg, W1h, W1t, b1r, W2, b2r, W3r, b3r)


def kernel(x, edge_index, head, tail, input, W_gnn, W_self, b_gnn,
           W1, b1, W2, b2, W3, b3):
    N, D = x.shape
    E = edge_index.shape[1]
    B = head.shape[0]
    C = 80
    assert E % (NW * C) == 0 and N % NS == 0 and (2 * B) % (NW * 64) == 0

    src2 = edge_index[0]
    dst2 = edge_index[1].reshape(NW, E // (NW * C), C)
    za = jnp.zeros((N, D), _f32)
    zd = jnp.zeros((N,), _f32)
    hp = jnp.concatenate([head, tail])
    (a0, a1, d0, d1, ga0, ga1, gx, gd0, gd1) = _make_agg_pair_kernel(
        N, D, E, 2 * B)(x, src2, dst2, za, zd, hp)

    bg = b_gnn.reshape(1, D)
    b1r = b1.reshape(1, D)
    b2r = b2.reshape(1, D)
    W3r = W3.reshape(1, D)
    b3r = b3.reshape(1, 1)
    return _mlp_call(B, D, ga0, ga1, gx, gd0, gd1,
                     W_gnn, W_self, bg, W1[:D], W1[D:], b1r, W2, b2r, W3r, b3r)
